# baseline (speedup 1.0000x reference)
import functools

import jax
import jax.numpy as jnp
from jax import lax
from jax.experimental import pallas as pl
from jax.experimental.pallas import tpu as pltpu
from jax.experimental.pallas import tpu_sc as plsc

N = 16384
EMB = 64
NUM_IN = 13
SEQ = 50
OUT = 128

NC = 2
NS = 16
NW = NC * NS
RPW = N // NW
CH = 128
NCH = RPW // CH

_sc_mesh = plsc.VectorSubcoreMesh(core_axis_name="c", subcore_axis_name="s")

K = 4
NBUF = 2 * K
NE = NCH * (SEQ - 1)


@functools.partial(
    pl.kernel,
    mesh=_sc_mesh,
    out_type=[
        jax.ShapeDtypeStruct((N, EMB), jnp.float32),
        jax.ShapeDtypeStruct((N // CH, CH), jnp.float32),
    ],
    scratch_types=[
        pltpu.VMEM((SEQ, RPW), jnp.int32),
        pltpu.VMEM((NCH, CH), jnp.int32),
        pltpu.VMEM((NCH, CH), jnp.float32),
        pltpu.VMEM((NBUF, CH, EMB), jnp.float32),
        pltpu.VMEM_SHARED((NS * NCH * CH, EMB), jnp.float32),
        pltpu.SemaphoreType.DMA,
        pltpu.SemaphoreType.DMA,
        pltpu.SemaphoreType.DMA,
        pltpu.SemaphoreType.DMA,
    ],
    compiler_params=pltpu.CompilerParams(use_tc_tiling_on_sc=False),
)
def _sc_encode(title_t, rix_all, e_text, tsum_out, lens_out,
               slab_v, rix_v, cnt_v, bufs, acc_sh, semg0, semg1, sems0, sems1):
    sid = lax.axis_index("s")
    wid = sid * NC + lax.axis_index("c")
    base = wid * RPW
    semg = (semg0, semg1)
    sems = (sems0, sems1)
    pltpu.sync_copy(title_t.at[:, pl.ds(base, RPW)], slab_v)
    pltpu.sync_copy(rix_all.at[pl.ds(sid * NCH, NCH)], rix_v)

    for c in range(NCH):
        pltpu.async_copy(e_text.at[slab_v.at[0, pl.ds(c * CH, CH)]],
                         bufs.at[0], semg0).wait()
        pltpu.sync_copy(bufs.at[0], acc_sh.at[pl.ds((sid * NCH + c) * CH, CH)])

    def fire_gathers(batch, grp):
        for b in range(K):
            e = batch * K + b
            c = e // (SEQ - 1)
            j = 1 + e - c * (SEQ - 1)
            pltpu.async_copy(e_text.at[slab_v.at[j, pl.ds(c * CH, CH)]],
                             bufs.at[grp * K + b], semg[grp])

    def wait_gathers(grp):
        for b in range(K):
            pltpu.make_async_copy(tsum_out.at[pl.ds(0, CH)],
                                  bufs.at[grp * K + b], semg[grp]).wait()

    def fire_scatters(batch, grp):
        for b in range(K):
            e = batch * K + b
            c = e // (SEQ - 1)
            pltpu.async_copy(bufs.at[grp * K + b], acc_sh.at[rix_v.at[c]],
                             sems[grp], add=True)

    def wait_scatters(grp):
        for b in range(K):
            pltpu.make_async_copy(tsum_out.at[pl.ds(0, CH)],
                                  bufs.at[grp * K + b], sems[grp]).wait()

    nbatch = NE // K
    fire_gathers(0, 0)

    zero = jnp.zeros((16,), jnp.float32)

    def count_body(j, cnts):
        return tuple(
            cnts[i] + jnp.where(slab_v[j, pl.ds(i * 16, 16)] != 0, 1.0, 0.0)
            for i in range(RPW // 16))

    cnts = lax.fori_loop(0, SEQ, count_body, (zero,) * (RPW // 16))
    for i in range(RPW // 16):
        cnt_v[i // 8, pl.ds((i % 8) * 16, 16)] = cnts[i]
    pltpu.sync_copy(cnt_v, lens_out.at[pl.ds(wid * NCH, NCH)])

    def body(k, carry):
        b0 = 2 * k
        wait_gathers(0)
        fire_scatters(b0, 0)
        fire_gathers(b0 + 1, 1)
        wait_scatters(0)
        wait_gathers(1)
        fire_scatters(b0 + 1, 1)
        fire_gathers(b0 + 2, 0)
        wait_scatters(1)
        return carry

    lax.fori_loop(0, (nbatch - 1) // 2, body, 0)
    wait_gathers(0)
    fire_scatters(nbatch - 1, 0)
    wait_scatters(0)
    pltpu.sync_copy(acc_sh.at[pl.ds(sid * NCH * CH, RPW)],
                    tsum_out.at[pl.ds(base, RPW)])


@functools.partial(
    pl.kernel,
    mesh=_sc_mesh,
    out_type=jax.ShapeDtypeStruct((N, EMB), jnp.float32),
    scratch_types=[
        pltpu.VMEM((NCH, CH), jnp.int32),
        pltpu.VMEM((NCH, CH), jnp.int32),
        pltpu.VMEM((2, CH, 2 * EMB), jnp.float32),
        pltpu.VMEM((CH, EMB), jnp.float32),
        pltpu.SemaphoreType.DMA,
        pltpu.SemaphoreType.DMA,
    ],
    compiler_params=pltpu.CompilerParams(
        use_tc_tiling_on_sc=True, needs_layout_passes=False),
)
def _sc_cat(item_r, e_cat2, cat_out, ids_v, blk_v, staged, outbuf, sem0, sem1):
    sid = lax.axis_index("s")
    wid = sid * NC + lax.axis_index("c")
    base = wid * RPW
    lane = jax.lax.broadcasted_iota(jnp.int32, (16,), 0)
    pltpu.sync_copy(item_r.at[pl.ds(wid * NCH, NCH)], ids_v)
    for p in range(RPW // 16):
        blk_v[p // 8, pl.ds((p % 8) * 16, 16)] = (
            ids_v[p // 8, pl.ds((p % 8) * 16, 16)] >> 1)

    def fire(c, grp):
        sem = (sem0, sem1)[grp]
        pltpu.async_copy(e_cat2.at[blk_v.at[c]], staged.at[grp], sem)

    def extract(c, grp):
        for p in range(CH // 16):
            vec = ids_v[c, pl.ds(p * 16, 16)]
            for q in range(16):
                s = p * 16 + q
                t = jax.lax.reduce_sum_p.bind(
                    jnp.where(lane == q, vec, 0), axes=(0,))
                r = t & 1
                for l in range(EMB // 16):
                    outbuf[s, pl.ds(l * 16, 16)] = (
                        staged[grp, s, pl.ds(r * EMB + l * 16, 16)])
        pltpu.sync_copy(outbuf, cat_out.at[pl.ds(base + c * CH, CH)])

    fire(0, 0)

    def cbody(c, carry):
        @pl.when(jnp.logical_and(c < NCH - 1, ((c + 1) & 1) == 0))
        def _():
            fire_c = c + 1
            pltpu.async_copy(e_cat2.at[blk_v.at[fire_c]], staged.at[0], sem0)

        @pl.when(jnp.logical_and(c < NCH - 1, ((c + 1) & 1) == 1))
        def _():
            fire_c = c + 1
            pltpu.async_copy(e_cat2.at[blk_v.at[fire_c]], staged.at[1], sem1)

        @pl.when((c & 1) == 0)
        def _():
            pltpu.make_async_copy(e_cat2.at[pl.ds(0, CH)], staged.at[0],
                                  sem0).wait()

        @pl.when((c & 1) == 1)
        def _():
            pltpu.make_async_copy(e_cat2.at[pl.ds(0, CH)], staged.at[1],
                                  sem1).wait()

        extract(c, c & 1)
        return carry

    lax.fori_loop(0, NCH, cbody, 0)


BN = 1024


def _tc_body(dense, lens, cat, tsum, w1, b1, w2, b2, wp, bp, out):
    h = jnp.maximum(
        jnp.dot(dense[...], w1[...], preferred_element_type=jnp.float32) + b1[...],
        0.0,
    )
    num = jnp.dot(h, w2[...], preferred_element_type=jnp.float32) + b2[...]
    lengths = jnp.maximum(lens[...], 1.0)
    pooled = tsum[...] / lengths
    wp_all = wp[...]
    r = jnp.dot(cat[...], wp_all[0:EMB], preferred_element_type=jnp.float32)
    r = r + jnp.dot(num, wp_all[EMB:2 * EMB], preferred_element_type=jnp.float32)
    r = r + jnp.dot(pooled, wp_all[2 * EMB:3 * EMB], preferred_element_type=jnp.float32)
    out[...] = r + bp[...]


_tc_combine = pl.pallas_call(
    _tc_body,
    grid=(N // BN,),
    in_specs=[
        pl.BlockSpec((BN, NUM_IN), lambda i: (i, 0)),
        pl.BlockSpec((BN, 1), lambda i: (i, 0)),
        pl.BlockSpec((BN, EMB), lambda i: (i, 0)),
        pl.BlockSpec((BN, EMB), lambda i: (i, 0)),
        pl.BlockSpec((NUM_IN, EMB), lambda i: (0, 0)),
        pl.BlockSpec((1, EMB), lambda i: (0, 0)),
        pl.BlockSpec((EMB, EMB), lambda i: (0, 0)),
        pl.BlockSpec((1, EMB), lambda i: (0, 0)),
        pl.BlockSpec((3 * EMB, OUT), lambda i: (0, 0)),
        pl.BlockSpec((1, OUT), lambda i: (0, 0)),
    ],
    out_specs=pl.BlockSpec((BN, OUT), lambda i: (i, 0)),
    out_shape=jax.ShapeDtypeStruct((N, OUT), jnp.float32),
)


def kernel(item_id, dense_feats, title, E_cat, W1, b1, W2, b2, E_text, Wp, bp):
    title32 = title.astype(jnp.int32)
    item32 = item_id.astype(jnp.int32)
    rix_all = jnp.arange(NS * NCH * CH, dtype=jnp.int32).reshape(NS * NCH, CH)
    tsum, lens128 = _sc_encode(title32.T, rix_all, E_text)
    lens = lens128.reshape(N, 1)
    cat_rows = _sc_cat(item32.reshape(NW * NCH, CH),
                       E_cat.reshape(E_cat.shape[0] // 2, 2 * EMB))
    return _tc_combine(
        dense_feats,
        lens,
        cat_rows,
        tsum,
        W1,
        b1.reshape(1, EMB),
        W2,
        b2.reshape(1, EMB),
        Wp,
        bp.reshape(1, OUT),
    )


# speedup vs baseline: 1.4051x; 1.4051x over previous
import functools

import jax
import jax.numpy as jnp
from jax import lax
from jax.experimental import pallas as pl
from jax.experimental.pallas import tpu as pltpu
from jax.experimental.pallas import tpu_sc as plsc

N = 16384
EMB = 64
NUM_IN = 13
SEQ = 50
OUT = 128

NC = 2
NS = 16
NW = NC * NS
RPW = N // NW
CH = 128
NCH = RPW // CH

_sc_mesh = plsc.VectorSubcoreMesh(core_axis_name="c", subcore_axis_name="s")

K = 4
NBUF = 2 * K
NE = NCH * (SEQ - 1)


@functools.partial(
    pl.kernel,
    mesh=_sc_mesh,
    out_type=[
        jax.ShapeDtypeStruct((N, EMB), jnp.float32),
        jax.ShapeDtypeStruct((N // CH, CH), jnp.float32),
    ],
    scratch_types=[
        pltpu.VMEM((SEQ, RPW), jnp.int32),
        pltpu.VMEM((NCH, CH), jnp.int32),
        pltpu.VMEM((NCH, CH), jnp.float32),
        pltpu.VMEM((NBUF, CH, EMB), jnp.float32),
        pltpu.VMEM_SHARED((NS * NCH * CH, EMB), jnp.float32),
        pltpu.SemaphoreType.DMA,
        pltpu.SemaphoreType.DMA,
        pltpu.SemaphoreType.DMA,
        pltpu.SemaphoreType.DMA,
    ],
    compiler_params=pltpu.CompilerParams(use_tc_tiling_on_sc=False),
)
def _sc_encode(title_t, rix_all, e_text, tsum_out, lens_out,
               slab_v, rix_v, cnt_v, bufs, acc_sh, semg0, semg1, sems0, sems1):
    sid = lax.axis_index("s")
    wid = sid * NC + lax.axis_index("c")
    base = wid * RPW
    semg = (semg0, semg1)
    sems = (sems0, sems1)
    pltpu.sync_copy(title_t.at[:, pl.ds(base, RPW)], slab_v)
    pltpu.sync_copy(rix_all.at[pl.ds(sid * NCH, NCH)], rix_v)

    for c in range(NCH):
        pltpu.async_copy(e_text.at[slab_v.at[0, pl.ds(c * CH, CH)]],
                         bufs.at[0], semg0).wait()
        pltpu.sync_copy(bufs.at[0], acc_sh.at[pl.ds((sid * NCH + c) * CH, CH)])

    def fire_gathers(batch, grp):
        for b in range(K):
            e = batch * K + b
            c = e // (SEQ - 1)
            j = 1 + e - c * (SEQ - 1)
            pltpu.async_copy(e_text.at[slab_v.at[j, pl.ds(c * CH, CH)]],
                             bufs.at[grp * K + b], semg[grp])

    def wait_gathers(grp):
        for b in range(K):
            pltpu.make_async_copy(tsum_out.at[pl.ds(0, CH)],
                                  bufs.at[grp * K + b], semg[grp]).wait()

    def fire_scatters(batch, grp):
        for b in range(K):
            e = batch * K + b
            c = e // (SEQ - 1)
            pltpu.async_copy(bufs.at[grp * K + b], acc_sh.at[rix_v.at[c]],
                             sems[grp], add=True)

    def wait_scatters(grp):
        for b in range(K):
            pltpu.make_async_copy(tsum_out.at[pl.ds(0, CH)],
                                  bufs.at[grp * K + b], sems[grp]).wait()

    nbatch = NE // K
    fire_gathers(0, 0)

    zero = jnp.zeros((16,), jnp.float32)

    def count_body(j, cnts):
        return tuple(
            cnts[i] + jnp.where(slab_v[j, pl.ds(i * 16, 16)] != 0, 1.0, 0.0)
            for i in range(RPW // 16))

    cnts = lax.fori_loop(0, SEQ, count_body, (zero,) * (RPW // 16))
    for i in range(RPW // 16):
        cnt_v[i // 8, pl.ds((i % 8) * 16, 16)] = cnts[i]
    pltpu.sync_copy(cnt_v, lens_out.at[pl.ds(wid * NCH, NCH)])

    def body(k, carry):
        b0 = 2 * k
        wait_gathers(0)
        fire_scatters(b0, 0)
        fire_gathers(b0 + 1, 1)
        wait_scatters(0)
        wait_gathers(1)
        fire_scatters(b0 + 1, 1)
        fire_gathers(b0 + 2, 0)
        wait_scatters(1)
        return carry

    lax.fori_loop(0, (nbatch - 1) // 2, body, 0)
    wait_gathers(0)
    fire_scatters(nbatch - 1, 0)
    wait_scatters(0)
    pltpu.sync_copy(acc_sh.at[pl.ds(sid * NCH * CH, RPW)],
                    tsum_out.at[pl.ds(base, RPW)])


CRING = 32
CNB = CH // CRING


@functools.partial(
    pl.kernel,
    mesh=_sc_mesh,
    out_type=jax.ShapeDtypeStruct((N, EMB), jnp.float32),
    scratch_types=[
        pltpu.VMEM((NCH, CH), jnp.int32),
        pltpu.VMEM((2, CRING, 8, EMB), jnp.float32),
        pltpu.VMEM((CH, EMB), jnp.float32),
        pltpu.SemaphoreType.DMA,
        pltpu.SemaphoreType.DMA,
    ],
    compiler_params=pltpu.CompilerParams(needs_layout_passes=False),
)
def _sc_cat(item_r, e_cat, cat_out, ids_v, staged, outbuf, sem0, sem1):
    sid = lax.axis_index("s")
    wid = sid * NC + lax.axis_index("c")
    base = wid * RPW
    sems = (sem0, sem1)
    lane = jax.lax.broadcasted_iota(jnp.int32, (16,), 0)
    pltpu.sync_copy(item_r.at[pl.ds(wid * NCH, NCH)], ids_v)

    def scalars(c, b):
        out = []
        for g in range(CRING // 16):
            vec = ids_v[c, pl.ds(b * CRING + g * 16, 16)]
            out.extend(
                jax.lax.reduce_sum_p.bind(
                    jnp.where(lane == s, vec, 0), axes=(0,))
                for s in range(16))
        return tuple(out)

    def fire(ts, grp):
        for s in range(CRING):
            t = ts[s]
            blk = pl.multiple_of(t - (t & 7), 8)
            pltpu.async_copy(e_cat.at[pl.ds(blk, 8)],
                             staged.at[grp, s], sems[grp])

    def drain(grp):
        for s in range(CRING):
            pltpu.make_async_copy(e_cat.at[pl.ds(0, 8)], staged.at[grp, s],
                                  sems[grp]).wait()

    def extract(ts, grp, obase):
        for s in range(CRING):
            r = ts[s] & 7
            for l in range(EMB // 16):
                outbuf[obase + s, pl.ds(l * 16, 16)] = (
                    staged[grp, s, r, pl.ds(l * 16, 16)])

    def cbody(c, carry):
        ts_a = scalars(c, 0)
        fire(ts_a, 0)

        def bbody(k, ts_a, c=c):
            b0 = 2 * k
            ts_b = scalars(c, b0 + 1)
            fire(ts_b, 1)
            drain(0)
            extract(ts_a, 0, b0 * CRING)
            ts_a2 = scalars(c, b0 + 2)
            fire(ts_a2, 0)
            drain(1)
            extract(ts_b, 1, (b0 + 1) * CRING)
            return ts_a2

        ts_a = lax.fori_loop(0, CNB // 2 - 1, bbody, ts_a)
        ts_b = scalars(c, CNB - 1)
        fire(ts_b, 1)
        drain(0)
        extract(ts_a, 0, (CNB - 2) * CRING)
        drain(1)
        extract(ts_b, 1, (CNB - 1) * CRING)
        pltpu.sync_copy(outbuf, cat_out.at[pl.ds(base + c * CH, CH)])
        return carry

    lax.fori_loop(0, NCH, cbody, 0)


BN = 1024


def _tc_body(dense, lens, cat, tsum, w1, b1, w2, b2, wp, bp, out):
    h = jnp.maximum(
        jnp.dot(dense[...], w1[...], preferred_element_type=jnp.float32) + b1[...],
        0.0,
    )
    num = jnp.dot(h, w2[...], preferred_element_type=jnp.float32) + b2[...]
    lengths = jnp.maximum(lens[...], 1.0)
    pooled = tsum[...] / lengths
    wp_all = wp[...]
    r = jnp.dot(cat[...], wp_all[0:EMB], preferred_element_type=jnp.float32)
    r = r + jnp.dot(num, wp_all[EMB:2 * EMB], preferred_element_type=jnp.float32)
    r = r + jnp.dot(pooled, wp_all[2 * EMB:3 * EMB], preferred_element_type=jnp.float32)
    out[...] = r + bp[...]


_tc_combine = pl.pallas_call(
    _tc_body,
    grid=(N // BN,),
    in_specs=[
        pl.BlockSpec((BN, NUM_IN), lambda i: (i, 0)),
        pl.BlockSpec((BN, 1), lambda i: (i, 0)),
        pl.BlockSpec((BN, EMB), lambda i: (i, 0)),
        pl.BlockSpec((BN, EMB), lambda i: (i, 0)),
        pl.BlockSpec((NUM_IN, EMB), lambda i: (0, 0)),
        pl.BlockSpec((1, EMB), lambda i: (0, 0)),
        pl.BlockSpec((EMB, EMB), lambda i: (0, 0)),
        pl.BlockSpec((1, EMB), lambda i: (0, 0)),
        pl.BlockSpec((3 * EMB, OUT), lambda i: (0, 0)),
        pl.BlockSpec((1, OUT), lambda i: (0, 0)),
    ],
    out_specs=pl.BlockSpec((BN, OUT), lambda i: (i, 0)),
    out_shape=jax.ShapeDtypeStruct((N, OUT), jnp.float32),
)


def kernel(item_id, dense_feats, title, E_cat, W1, b1, W2, b2, E_text, Wp, bp):
    title32 = title.astype(jnp.int32)
    item32 = item_id.astype(jnp.int32)
    rix_all = jnp.arange(NS * NCH * CH, dtype=jnp.int32).reshape(NS * NCH, CH)
    tsum, lens128 = _sc_encode(title32.T, rix_all, E_text)
    lens = lens128.reshape(N, 1)
    cat_rows = _sc_cat(item32.reshape(NW * NCH, CH), E_cat)
    return _tc_combine(
        dense_feats,
        lens,
        cat_rows,
        tsum,
        W1,
        b1.reshape(1, EMB),
        W2,
        b2.reshape(1, EMB),
        Wp,
        bp.reshape(1, OUT),
    )
